# Initial kernel scaffold; baseline (speedup 1.0000x reference)
import jax
import jax.numpy as jnp
from jax.experimental import pallas as pl


def kernel(user_emb, uploader_emb, fc_W, fc_b, fcg_W, fcg_b, edge_vals, edge_index):
    raise NotImplementedError("write your pallas kernel here")


# speedup vs baseline: 2.4945x; 2.4945x over previous
import functools
import jax
import jax.numpy as jnp
from jax import lax
from jax.experimental import pallas as pl
from jax.experimental.pallas import tpu as pltpu
from jax.experimental.pallas import tpu_sc as plsc

NU = 50000
NN = 60000
DD = 64
EE = 960000
KE = 128
SR = 128
EP = 962560
TB = 600

_MESH = dict(core_axis_name="c", subcore_axis_name="s")


def _spmm_body(gsel, bnd_hbm, rows_hbm, cols_hbm, vals_hbm, x2_hbm, y_hbm,
               bnd_v, rowv, colv, valv, pairv, xbuf, stag, gsem):
    c = lax.axis_index("c")
    s = lax.axis_index("s")
    t = c * 16 + s
    pltpu.sync_copy(bnd_hbm, bnd_v)
    vb = bnd_v[pl.ds(t, 16)]
    start_raw = vb[0]
    end_raw = vb[1]
    zb = bnd_v[pl.ds(33 + t, 16)]
    z0 = zb[0]
    rend = zb[1]
    start = start_raw & jnp.int32(-8)
    nch = jnp.maximum(end_raw - start + (KE - 1), 0) // KE

    iota = lax.broadcasted_iota(jnp.int32, (16,), 0)
    zero16 = jnp.zeros((16,), jnp.float32)

    def zstag(i, _):
        stag[pl.ds(i * 16, 16)] = zero16
        return 0
    lax.fori_loop(0, SR * DD // 16, zstag, 0)

    def drains(prev, sb):
        nd = (prev - sb) // SR
        def dbody(j, _):
            pltpu.sync_copy(stag, y_hbm.at[pl.ds((sb + j * SR) * DD,
                                                 SR * DD)])
            for i in range(SR * DD // 16):
                stag[pl.ds(i * 16, 16)] = zero16
            return 0
        lax.fori_loop(0, nd, dbody, 0)
        return sb + nd * SR

    def chunk(i, carry):
        base = pl.multiple_of(start + i * KE, 8)
        pltpu.sync_copy(rows_hbm.at[pl.ds(base, KE)], rowv.at[pl.ds(0, KE)])
        pltpu.sync_copy(cols_hbm.at[pl.ds(base, KE)], colv.at[pl.ds(0, KE)])
        pltpu.sync_copy(vals_hbm.at[pl.ds(base, KE)], valv)
        for q in range(KE // 16):
            pairv[pl.ds(16 * q, 16)] = colv[pl.ds(16 * q, 16)] >> 1
        pltpu.async_copy(x2_hbm.at[pairv], xbuf, gsem).wait()

        def edge(k, cr):
            a0, a1, a2, a3, prev, sb = cr
            ev = base + k
            inb = (ev >= start_raw) & (ev < end_raw)
            rs = rowv[pl.ds(k, 16)]
            cs = colv[pl.ds(k, 16)]
            vsl = valv[k, pl.ds(0, 16)]
            r_eff = jnp.where(inb, rs[0], prev)
            v_eff = jnp.where(inb, vsl[gsel], 0.0)
            off = (cs[0] & 1) * DD
            vv = jnp.full((16,), v_eff, jnp.float32)
            x0 = xbuf[k, pl.ds(off, 16)] * vv
            x1 = xbuf[k, pl.ds(off + 16, 16)] * vv
            x2 = xbuf[k, pl.ds(off + 32, 16)] * vv
            x3 = xbuf[k, pl.ds(off + 48, 16)] * vv
            sb = drains(prev, sb)
            o = (prev - sb) * DD
            stag[pl.ds(o, 16)] = a0
            stag[pl.ds(o + 16, 16)] = a1
            stag[pl.ds(o + 32, 16)] = a2
            stag[pl.ds(o + 48, 16)] = a3
            keep = jnp.where(r_eff != prev, 0.0, 1.0)
            kv = jnp.full((16,), keep, jnp.float32)
            a0 = a0 * kv + x0
            a1 = a1 * kv + x1
            a2 = a2 * kv + x2
            a3 = a3 * kv + x3
            return (a0, a1, a2, a3, r_eff, sb)
        return lax.fori_loop(0, KE, edge, carry)

    rb = bnd_v[pl.ds(66 + t, 16)]
    has = start_raw < end_raw
    first_row = jnp.where(has, rb[0], z0)
    carry0 = (zero16, zero16, zero16, zero16, first_row, z0)
    a0, a1, a2, a3, prev, sb = lax.fori_loop(0, nch, chunk, carry0)

    sb = drains(prev, sb)
    o = (prev - sb) * DD
    stag[pl.ds(o, 16)] = a0
    stag[pl.ds(o + 16, 16)] = a1
    stag[pl.ds(o + 32, 16)] = a2
    stag[pl.ds(o + 48, 16)] = a3
    sb = drains(rend, sb)

    def tail(i, _):
        pltpu.sync_copy(stag.at[pl.ds(i * DD, DD)],
                        y_hbm.at[pl.ds((sb + i) * DD, DD)])
        return 0
    lax.fori_loop(0, jnp.maximum(rend - sb, 0), tail, 0)


def _spmm(bnd, rows, cols, vals16, gsel, x):
    x2 = x.reshape(NN // 2, 2 * DD)
    f = pl.kernel(
        functools.partial(_spmm_body, gsel),
        mesh=plsc.VectorSubcoreMesh(**_MESH),
        out_type=jax.ShapeDtypeStruct((NN * DD,), jnp.float32),
        scratch_types=[
            pltpu.VMEM((128,), jnp.int32),
            pltpu.VMEM((KE + 16,), jnp.int32),
            pltpu.VMEM((KE + 16,), jnp.int32),
            pltpu.VMEM((KE, 16), jnp.float32),
            pltpu.VMEM((KE,), jnp.int32),
            pltpu.VMEM((KE, 2 * DD), jnp.float32),
            pltpu.VMEM((SR * DD,), jnp.float32),
            pltpu.SemaphoreType.DMA,
        ],
    )
    return f(bnd, rows, cols, vals16, x2).reshape(NN, DD)


def _subv_body(rows_hbm, cols_hbm, vals_hbm, oh8_hbm, out_hbm,
               rowv, colv, valv, r8v, c8v, ohr, ohc, obuf, sem):
    c = lax.axis_index("c")
    s = lax.axis_index("s")
    w = c * 16 + s
    per_tile = EP // 32
    nch = per_tile // KE
    def chunk(i, _):
        base = w * per_tile + i * KE
        pltpu.sync_copy(rows_hbm.at[pl.ds(base, KE)], rowv.at[pl.ds(0, KE)])
        pltpu.sync_copy(cols_hbm.at[pl.ds(base, KE)], colv.at[pl.ds(0, KE)])
        pltpu.sync_copy(vals_hbm.at[pl.ds(base, KE)], valv.at[pl.ds(0, KE)])
        for q in range(KE // 16):
            r8v[pl.ds(16 * q, 16)] = rowv[pl.ds(16 * q, 16)] >> 3
            c8v[pl.ds(16 * q, 16)] = colv[pl.ds(16 * q, 16)] >> 3
        pltpu.async_copy(oh8_hbm.at[r8v], ohr, sem).wait()
        pltpu.async_copy(oh8_hbm.at[c8v], ohc, sem).wait()
        def sed(k, _):
            rs = rowv[pl.ds(k, 16)]
            cs = colv[pl.ds(k, 16)]
            vs = valv[pl.ds(k, 16)]
            vv = jnp.full((16,), vs[0], jnp.float32)
            er = ohr[k, pl.ds((rs[0] & 7) * 16, 16)]
            ec = ohc[k, pl.ds((cs[0] & 7) * 16, 16)]
            obuf[k, pl.ds(0, 16)] = er * ec * vv
            return 0
        lax.fori_loop(0, KE, sed, 0)
        pltpu.sync_copy(obuf, out_hbm.at[pl.ds(base, KE)])
        return 0
    lax.fori_loop(0, nch, chunk, 0)


def _subv(rows, cols, vals, oh):
    oh8 = oh.reshape(NN // 8, 128)
    f = pl.kernel(
        _subv_body,
        mesh=plsc.VectorSubcoreMesh(**_MESH),
        out_type=jax.ShapeDtypeStruct((EP, 16), jnp.float32),
        scratch_types=[
            pltpu.VMEM((KE + 16,), jnp.int32),
            pltpu.VMEM((KE + 16,), jnp.int32),
            pltpu.VMEM((KE + 16,), jnp.float32),
            pltpu.VMEM((KE,), jnp.int32),
            pltpu.VMEM((KE,), jnp.int32),
            pltpu.VMEM((KE, 128), jnp.float32),
            pltpu.VMEM((KE, 128), jnp.float32),
            pltpu.VMEM((KE, 16), jnp.float32),
            pltpu.SemaphoreType.DMA,
        ],
    )
    return f(rows, cols, vals, oh8)


def _route_body(ego, side, W, b, Wg, bg, oh):
    t = ego[...] + side[...]
    h = jnp.dot(t, W[...], preferred_element_type=jnp.float32) + b[...]
    h = jnp.where(h >= 0, h, 0.01 * h)
    sc = jnp.dot(h, Wg[...], preferred_element_type=jnp.float32) + bg[...]
    m = jnp.max(sc, axis=1, keepdims=True)
    ohf = (sc == m).astype(jnp.float32)
    pid = pl.program_id(0)
    rid = pid * TB + lax.broadcasted_iota(jnp.int32, (TB, 128), 0)
    lanes = lax.broadcasted_iota(jnp.int32, (TB, 128), 1)
    up_oh = jnp.where(lanes < 4, 1.0, 0.0).astype(jnp.float32)
    oh128 = jnp.where(rid >= NU, up_oh, ohf)
    oh[...] = oh128[:, :16]


def _route(ego, side, W, b, Wg, bg):
    return pl.pallas_call(
        _route_body,
        grid=(NN // TB,),
        in_specs=[
            pl.BlockSpec((TB, DD), lambda i: (i, 0)),
            pl.BlockSpec((TB, DD), lambda i: (i, 0)),
            pl.BlockSpec((DD, 128), lambda i: (0, 0)),
            pl.BlockSpec((1, 128), lambda i: (0, 0)),
            pl.BlockSpec((128, 128), lambda i: (0, 0)),
            pl.BlockSpec((1, 128), lambda i: (0, 0)),
        ],
        out_specs=pl.BlockSpec((TB, 16), lambda i: (i, 0)),
        out_shape=jax.ShapeDtypeStruct((NN, 16), jnp.float32),
    )(ego, side, W, b, Wg, bg)


def _acc4_body(a, b, c, d, o):
    o[...] = a[...] + b[...] + c[...] + d[...]


def _acc5_body(p, a, b, c, d, o):
    o[...] = p[...] + a[...] + b[...] + c[...] + d[...]


def _final_body(p, a, b, c, d, e, o):
    o[...] = 0.2 * (p[...] + a[...] + b[...] + c[...] + d[...]) + 0.8 * e[...]


def _ew_call(body, n_in, args):
    return pl.pallas_call(
        body,
        grid=(NN // TB,),
        in_specs=[pl.BlockSpec((TB, DD), lambda i: (i, 0))] * n_in,
        out_specs=pl.BlockSpec((TB, DD), lambda i: (i, 0)),
        out_shape=jax.ShapeDtypeStruct((NN, DD), jnp.float32),
    )(*args)


def kernel(user_emb, uploader_emb, fc_W, fc_b, fcg_W, fcg_b, edge_vals,
           edge_index):
    rows = edge_index[0].astype(jnp.int32)
    cols = edge_index[1].astype(jnp.int32)
    order = jnp.argsort(rows)
    rows_s = jnp.concatenate(
        [rows[order], jnp.full((EP - EE,), NN - 1, jnp.int32)])
    cols_s = jnp.concatenate([cols[order], jnp.zeros((EP - EE,), jnp.int32)])
    vals_s = jnp.concatenate(
        [edge_vals[order], jnp.zeros((EP - EE,), jnp.float32)])

    t32 = jnp.arange(32, dtype=jnp.int32)
    pos = (t32 * EE) // 32
    starts = jnp.searchsorted(
        rows_s[:EE], rows_s[pos], side='left').astype(jnp.int32)
    starts = jnp.concatenate([starts, jnp.full((1,), EE, jnp.int32)])
    zrows = rows_s[starts[:32]]
    zrows = jnp.concatenate([
        jnp.zeros((1,), jnp.int32), zrows[1:],
        jnp.full((1,), NN, jnp.int32)])
    frows = jnp.concatenate([rows_s[starts[:32]],
                             jnp.full((1,), NN - 1, jnp.int32)])
    bnd = jnp.concatenate(
        [starts, zrows, frows, jnp.zeros((29,), jnp.int32)])

    all_emb = jnp.concatenate([user_emb, uploader_emb], axis=0)
    vals16 = jnp.pad(vals_s[:, None], ((0, 0), (0, 15)))
    side = _spmm(bnd, rows_s, cols_s, vals16, 0, all_emb)

    W_pad = jnp.pad(fc_W, ((0, 0), (0, 64)))
    b_pad = jnp.pad(fc_b, (0, 64)).reshape(1, 128)
    Wg_pad = jnp.pad(fcg_W, ((0, 64), (0, 124)))
    bg_pad = jnp.concatenate(
        [fcg_b, jnp.full((124,), -1e30, jnp.float32)]).reshape(1, 128)
    oh = _route(all_emb, side, W_pad, b_pad, Wg_pad, bg_pad)

    subs = _subv(rows_s, cols_s, vals_s, oh)
    xs = [all_emb] * 4
    acc = None
    out = None
    for k in range(1, 5):
        ys = [_spmm(bnd, rows_s, cols_s, subs, g, xs[g]) for g in range(4)]
        if k == 1:
            acc = _ew_call(_acc4_body, 4, ys)
        elif k < 4:
            acc = _ew_call(_acc5_body, 5, [acc] + ys)
        else:
            out = _ew_call(_final_body, 6, [acc] + ys + [all_emb])
        xs = ys
    return out[:NU], out[NU:]
